# Initial kernel scaffold; baseline (speedup 1.0000x reference)
import jax
import jax.numpy as jnp
from jax.experimental import pallas as pl


def kernel(x, edge_index, W1, b1, W2, b2):
    raise NotImplementedError("write your pallas kernel here")


# speedup vs baseline: 12.5061x; 12.5061x over previous
import functools

import jax
import jax.numpy as jnp
from jax import lax
from jax.experimental import pallas as pl
from jax.experimental.pallas import tpu as pltpu
from jax.experimental.pallas import tpu_sc as plsc

N = 10000
NP = 10240
E = 320000
D = 128

NC = 2
NS = 16
NW = NC * NS
EPW = E // NW
K = 80
CH = EPW // K
RPS = NP // NS
ZR = 32

_f32 = jnp.float32


def _sc_mesh():
    return plsc.VectorSubcoreMesh(core_axis_name="c", subcore_axis_name="s")


def _deg_partials(dst):
    @functools.partial(
        pl.kernel,
        out_type=jax.ShapeDtypeStruct((NC, NP, D), _f32),
        mesh=_sc_mesh(),
        scratch_types=[
            pltpu.VMEM((K,), jnp.int32),
            pltpu.VMEM((K, D), _f32),
            pltpu.VMEM((ZR, D), _f32),
            pltpu.VMEM_SHARED((NP, D), _f32),
        ],
    )
    def k(dst_hbm, out_hbm, idx_v, ones_v, zero_v, acc_sh):
        c = lax.axis_index("c")
        s = lax.axis_index("s")
        wid = c * NS + s
        zvec = jnp.zeros((16,), _f32)
        ovec = jnp.full((16,), 1.0, _f32)

        @pl.loop(0, ZR)
        def _(r):
            @pl.loop(0, D // 16)
            def _(j):
                zero_v[r, pl.ds(j * 16, 16)] = zvec

        @pl.loop(0, K)
        def _(r):
            @pl.loop(0, D // 16)
            def _(j):
                ones_v[r, pl.ds(j * 16, 16)] = ovec

        @pl.loop(0, RPS // ZR)
        def _(t):
            pltpu.sync_copy(zero_v, acc_sh.at[pl.ds(s * RPS + t * ZR, ZR)])

        plsc.subcore_barrier()

        base0 = wid * EPW

        @pl.loop(0, CH)
        def _(i):
            pltpu.sync_copy(dst_hbm.at[pl.ds(base0 + i * K, K)], idx_v)
            pltpu.sync_copy(ones_v, acc_sh.at[idx_v], add=True)

        plsc.subcore_barrier()
        pltpu.sync_copy(
            acc_sh.at[pl.ds(s * RPS, RPS)], out_hbm.at[c, pl.ds(s * RPS, RPS)]
        )

    return k(dst)


def _msg_partials(g, src, dst):
    @functools.partial(
        pl.kernel,
        out_type=jax.ShapeDtypeStruct((NC, NP, D), _f32),
        mesh=_sc_mesh(),
        scratch_types=[
            pltpu.VMEM((K,), jnp.int32),
            pltpu.VMEM((K,), jnp.int32),
            pltpu.VMEM((K, D), _f32),
            pltpu.VMEM((ZR, D), _f32),
            pltpu.VMEM_SHARED((NP, D), _f32),
            pltpu.SemaphoreType.DMA,
        ],
    )
    def k(g_hbm, src_hbm, dst_hbm, out_hbm, src_v, dst_v, rows_v, zero_v, acc_sh, sem):
        c = lax.axis_index("c")
        s = lax.axis_index("s")
        wid = c * NS + s
        zvec = jnp.zeros((16,), _f32)

        @pl.loop(0, ZR)
        def _(r):
            @pl.loop(0, D // 16)
            def _(j):
                zero_v[r, pl.ds(j * 16, 16)] = zvec

        @pl.loop(0, RPS // ZR)
        def _(t):
            pltpu.sync_copy(zero_v, acc_sh.at[pl.ds(s * RPS + t * ZR, ZR)])

        plsc.subcore_barrier()

        base0 = wid * EPW

        @pl.loop(0, CH)
        def _(i):
            base = base0 + i * K
            pltpu.sync_copy(src_hbm.at[pl.ds(base, K)], src_v)
            pltpu.sync_copy(dst_hbm.at[pl.ds(base, K)], dst_v)
            pltpu.async_copy(g_hbm.at[src_v], rows_v, sem).wait()
            pltpu.sync_copy(rows_v, acc_sh.at[dst_v], add=True)

        plsc.subcore_barrier()
        pltpu.sync_copy(
            acc_sh.at[pl.ds(s * RPS, RPS)], out_hbm.at[c, pl.ds(s * RPS, RPS)]
        )

    return k(g, src, dst)


def _dinv_col(degp_a, degp_b):
    return jax.lax.rsqrt(degp_a + degp_b + 1.0)[:, 0:1]


def _mm(x, W):
    def body(x_ref, w_ref, o_ref):
        o_ref[...] = jnp.dot(x_ref[...], w_ref[...], preferred_element_type=_f32)

    return pl.pallas_call(
        body, out_shape=jax.ShapeDtypeStruct((N, D), _f32)
    )(x, W)


def _scale(h, degp):
    def body(h_ref, degp_ref, o_ref):
        dinv = _dinv_col(degp_ref[0, :N], degp_ref[1, :N])
        o_ref[...] = h_ref[...] * dinv

    return pl.pallas_call(
        body, out_shape=jax.ShapeDtypeStruct((N, D), _f32)
    )(h, degp)


def _mid(p, g1, degp, b1, W2):
    def body(p_ref, g1_ref, degp_ref, b_ref, w_ref, o_ref):
        dinv = _dinv_col(degp_ref[0, :N], degp_ref[1, :N])
        x2 = jnp.maximum(
            dinv * (p_ref[0, :N] + p_ref[1, :N] + g1_ref[...]) + b_ref[...], 0.0
        )
        o_ref[...] = jnp.dot(x2, w_ref[...], preferred_element_type=_f32) * dinv

    return pl.pallas_call(
        body, out_shape=jax.ShapeDtypeStruct((N, D), _f32)
    )(p, g1, degp, b1.reshape(1, D), W2)


def _final(p, g2, degp, b2):
    def body(p_ref, g2_ref, degp_ref, b_ref, o_ref):
        dinv = _dinv_col(degp_ref[0, :N], degp_ref[1, :N])
        o_ref[...] = jnp.maximum(
            dinv * (p_ref[0, :N] + p_ref[1, :N] + g2_ref[...]) + b_ref[...], 0.0
        )

    return pl.pallas_call(
        body, out_shape=jax.ShapeDtypeStruct((N, D), _f32)
    )(p, g2, degp, b2.reshape(1, D))


def kernel(x, edge_index, W1, b1, W2, b2):
    src = edge_index[0]
    dst = edge_index[1]

    degp = _deg_partials(dst)
    h1 = _mm(x, W1)
    g1 = _scale(h1, degp)
    p1 = _msg_partials(g1, src, dst)
    g2 = _mid(p1, g1, degp, b1, W2)
    p2 = _msg_partials(g2, src, dst)
    out = _final(p2, g2, degp, b2)
    return out


# speedup vs baseline: 21.5690x; 1.7247x over previous
import functools

import jax
import jax.numpy as jnp
from jax import lax
from jax.experimental import pallas as pl
from jax.experimental.pallas import tpu as pltpu
from jax.experimental.pallas import tpu_sc as plsc

N = 10000
NP = 10240
E = 320000
D = 128

NC = 2
NS = 16
NW = NC * NS
K = 80
CH = 128
EPAD = NW * CH * K
FB = 8
RPS = NP // NS
ZR = 8

_f32 = jnp.float32


def _sc_mesh():
    return plsc.VectorSubcoreMesh(core_axis_name="c", subcore_axis_name="s")


def _edges_interleaved(src, dst):
    pad = EPAD - E
    ar = jnp.arange(pad, dtype=jnp.int32)
    src_f = jnp.concatenate([src, (ar * 131) % N])
    dst_f = jnp.concatenate([dst, N + (ar % (NP - N))])
    return jnp.stack(
        [src_f.reshape(NW, CH, K), dst_f.reshape(NW, CH, K)], axis=2
    )


def _deg_partials(er):
    @functools.partial(
        pl.kernel,
        out_type=jax.ShapeDtypeStruct((NC, NP, D), _f32),
        mesh=_sc_mesh(),
        scratch_types=[
            pltpu.VMEM((CH, 2, K), jnp.int32),
            pltpu.VMEM((K, D), _f32),
            pltpu.VMEM((ZR, D), _f32),
            pltpu.VMEM_SHARED((NP, D), _f32),
            pltpu.SemaphoreType.DMA,
        ],
    )
    def k(er_hbm, out_hbm, idx_all, ones_v, zero_v, acc_sh, sem):
        c = lax.axis_index("c")
        s = lax.axis_index("s")
        wid = c * NS + s
        pltpu.sync_copy(er_hbm.at[wid], idx_all)
        zvec = jnp.zeros((16,), _f32)
        ovec = jnp.full((16,), 1.0, _f32)

        @pl.loop(0, ZR)
        def _(r):
            @pl.loop(0, D // 16)
            def _(j):
                zero_v[r, pl.ds(j * 16, 16)] = zvec

        @pl.loop(0, K)
        def _(r):
            @pl.loop(0, D // 16)
            def _(j):
                ones_v[r, pl.ds(j * 16, 16)] = ovec

        @pl.loop(0, RPS // ZR)
        def _(t):
            pltpu.sync_copy(zero_v, acc_sh.at[pl.ds(s * RPS + t * ZR, ZR)])

        plsc.subcore_barrier()

        @pl.loop(0, CH, step=FB)
        def _(i0):
            for b in range(FB):
                pltpu.async_copy(
                    ones_v, acc_sh.at[idx_all.at[i0 + b, 1]], sem, add=True
                )
            for b in range(FB):
                pltpu.make_async_copy(
                    ones_v, acc_sh.at[idx_all.at[i0 + b, 1]], sem
                ).wait()

        plsc.subcore_barrier()
        pltpu.sync_copy(
            acc_sh.at[pl.ds(s * RPS, RPS)], out_hbm.at[c, pl.ds(s * RPS, RPS)]
        )

    return k(er)


def _msg_partials(g, er):
    @functools.partial(
        pl.kernel,
        out_type=jax.ShapeDtypeStruct((NC, NP, D), _f32),
        mesh=_sc_mesh(),
        scratch_types=[
            pltpu.VMEM((4, 2, K), jnp.int32),
            pltpu.VMEM((2, K, D), _f32),
            pltpu.VMEM((ZR, D), _f32),
            pltpu.VMEM_SHARED((NP, D), _f32),
        ] + [pltpu.SemaphoreType.DMA] * 6,
    )
    def k(g_hbm, er_hbm, out_hbm, idx, rows, zero_v, acc_sh, *sems):
        isem = sems[:4]
        gsem = sems[4:]
        c = lax.axis_index("c")
        s = lax.axis_index("s")
        wid = c * NS + s
        zvec = jnp.zeros((16,), _f32)

        @pl.loop(0, ZR)
        def _(r):
            @pl.loop(0, D // 16)
            def _(j):
                zero_v[r, pl.ds(j * 16, 16)] = zvec

        @pl.loop(0, RPS // ZR)
        def _(t):
            pltpu.sync_copy(zero_v, acc_sh.at[pl.ds(s * RPS + t * ZR, ZR)])

        plsc.subcore_barrier()

        for j in range(4):
            pltpu.async_copy(er_hbm.at[wid, j], idx.at[j], isem[j])
        pltpu.make_async_copy(er_hbm.at[wid, 0], idx.at[0], isem[0]).wait()
        pltpu.async_copy(g_hbm.at[idx.at[0, 0]], rows.at[0], gsem[0])

        @pl.loop(0, CH, step=4)
        def _(i0):
            for u in range(4):
                i = i0 + u
                b = u % 2
                nb = (u + 1) % 2
                pltpu.make_async_copy(
                    g_hbm.at[idx.at[u, 0]], rows.at[b], gsem[b]
                ).wait()

                @pl.when(i + 1 < CH)
                def _():
                    pltpu.make_async_copy(
                        er_hbm.at[wid, i + 1], idx.at[(u + 1) % 4],
                        isem[(u + 1) % 4],
                    ).wait()
                    pltpu.async_copy(
                        g_hbm.at[idx.at[(u + 1) % 4, 0]], rows.at[nb],
                        gsem[nb],
                    )

                pltpu.sync_copy(rows.at[b], acc_sh.at[idx.at[u, 1]], add=True)

                @pl.when(i + 4 < CH)
                def _():
                    pltpu.async_copy(er_hbm.at[wid, i + 4], idx.at[u], isem[u])

        plsc.subcore_barrier()
        pltpu.sync_copy(
            acc_sh.at[pl.ds(s * RPS, RPS)], out_hbm.at[c, pl.ds(s * RPS, RPS)]
        )

    return k(g, er)


def _dinv_col(degp_a, degp_b):
    return jax.lax.rsqrt(degp_a + degp_b + 1.0)[:, 0:1]


def _mm(x, W):
    def body(x_ref, w_ref, o_ref):
        o_ref[...] = jnp.dot(x_ref[...], w_ref[...], preferred_element_type=_f32)

    return pl.pallas_call(
        body, out_shape=jax.ShapeDtypeStruct((N, D), _f32)
    )(x, W)


def _scale(h, degp):
    def body(h_ref, degp_ref, o_ref):
        dinv = _dinv_col(degp_ref[0, :N], degp_ref[1, :N])
        o_ref[...] = h_ref[...] * dinv

    return pl.pallas_call(
        body, out_shape=jax.ShapeDtypeStruct((N, D), _f32)
    )(h, degp)


def _mid(p, g1, degp, b1, W2):
    def body(p_ref, g1_ref, degp_ref, b_ref, w_ref, o_ref):
        dinv = _dinv_col(degp_ref[0, :N], degp_ref[1, :N])
        x2 = jnp.maximum(
            dinv * (p_ref[0, :N] + p_ref[1, :N] + g1_ref[...]) + b_ref[...], 0.0
        )
        o_ref[...] = jnp.dot(x2, w_ref[...], preferred_element_type=_f32) * dinv

    return pl.pallas_call(
        body, out_shape=jax.ShapeDtypeStruct((N, D), _f32)
    )(p, g1, degp, b1.reshape(1, D), W2)


def _final(p, g2, degp, b2):
    def body(p_ref, g2_ref, degp_ref, b_ref, o_ref):
        dinv = _dinv_col(degp_ref[0, :N], degp_ref[1, :N])
        o_ref[...] = jnp.maximum(
            dinv * (p_ref[0, :N] + p_ref[1, :N] + g2_ref[...]) + b_ref[...], 0.0
        )

    return pl.pallas_call(
        body, out_shape=jax.ShapeDtypeStruct((N, D), _f32)
    )(p, g2, degp, b2.reshape(1, D))


def kernel(x, edge_index, W1, b1, W2, b2):
    er = _edges_interleaved(edge_index[0], edge_index[1])

    degp = _deg_partials(er)
    h1 = _mm(x, W1)
    g1 = _scale(h1, degp)
    p1 = _msg_partials(g1, er)
    g2 = _mid(p1, g1, degp, b1, W2)
    p2 = _msg_partials(g2, er)
    out = _final(p2, g2, degp, b2)
    return out


# speedup vs baseline: 24.8628x; 1.1527x over previous
import functools

import jax
import jax.numpy as jnp
from jax import lax
from jax.experimental import pallas as pl
from jax.experimental.pallas import tpu as pltpu
from jax.experimental.pallas import tpu_sc as plsc

N = 10000
NP = 10240
E = 320000
D = 128

NC = 2
NS = 16
NW = NC * NS
K = 128
CH = 80
EPAD = NW * CH * K
FB = 8
RPS = NP // NS
ZR = 8

_f32 = jnp.float32


def _sc_mesh():
    return plsc.VectorSubcoreMesh(core_axis_name="c", subcore_axis_name="s")


def _edges_interleaved(src, dst):
    pad = EPAD - E
    ar = jnp.arange(pad, dtype=jnp.int32)
    src_f = jnp.concatenate([src, (ar * 131) % N])
    dst_f = jnp.concatenate([dst, N + (ar % (NP - N))])
    return jnp.stack(
        [src_f.reshape(NW, CH, K), dst_f.reshape(NW, CH, K)], axis=2
    )


def _deg_partials(er):
    @functools.partial(
        pl.kernel,
        out_type=jax.ShapeDtypeStruct((NC, NP, D), _f32),
        mesh=_sc_mesh(),
        scratch_types=[
            pltpu.VMEM((CH, 2, K), jnp.int32),
            pltpu.VMEM((K, D), _f32),
            pltpu.VMEM((ZR, D), _f32),
            pltpu.VMEM_SHARED((NP, D), _f32),
            pltpu.SemaphoreType.DMA,
        ],
    )
    def k(er_hbm, out_hbm, idx_all, ones_v, zero_v, acc_sh, sem):
        c = lax.axis_index("c")
        s = lax.axis_index("s")
        wid = c * NS + s
        pltpu.sync_copy(er_hbm.at[wid], idx_all)
        zvec = jnp.zeros((16,), _f32)
        ovec = jnp.full((16,), 1.0, _f32)

        @pl.loop(0, ZR)
        def _(r):
            @pl.loop(0, D // 16)
            def _(j):
                zero_v[r, pl.ds(j * 16, 16)] = zvec

        @pl.loop(0, K)
        def _(r):
            @pl.loop(0, D // 16)
            def _(j):
                ones_v[r, pl.ds(j * 16, 16)] = ovec

        @pl.loop(0, RPS // ZR)
        def _(t):
            pltpu.sync_copy(zero_v, acc_sh.at[pl.ds(s * RPS + t * ZR, ZR)])

        plsc.subcore_barrier()

        @pl.loop(0, CH, step=FB)
        def _(i0):
            for b in range(FB):
                pltpu.async_copy(
                    ones_v, acc_sh.at[idx_all.at[i0 + b, 1]], sem, add=True
                )
            for b in range(FB):
                pltpu.make_async_copy(
                    ones_v, acc_sh.at[idx_all.at[i0 + b, 1]], sem
                ).wait()

        plsc.subcore_barrier()
        pltpu.sync_copy(
            acc_sh.at[pl.ds(s * RPS, RPS)], out_hbm.at[c, pl.ds(s * RPS, RPS)]
        )

    return k(er)


def _msg_partials(g, er):
    @functools.partial(
        pl.kernel,
        out_type=jax.ShapeDtypeStruct((NC, NP, D), _f32),
        mesh=_sc_mesh(),
        scratch_types=[
            pltpu.VMEM((4, 2, K), jnp.int32),
            pltpu.VMEM((2, K, D), _f32),
            pltpu.VMEM((ZR, D), _f32),
            pltpu.VMEM_SHARED((NP, D), _f32),
        ] + [pltpu.SemaphoreType.DMA] * 8,
    )
    def k(g_hbm, er_hbm, out_hbm, idx, rows, zero_v, acc_sh, *sems):
        isem = sems[:4]
        gsem = sems[4:6]
        ssem = sems[6:8]
        c = lax.axis_index("c")
        s = lax.axis_index("s")
        wid = c * NS + s
        zvec = jnp.zeros((16,), _f32)

        @pl.loop(0, ZR)
        def _(r):
            @pl.loop(0, D // 16)
            def _(j):
                zero_v[r, pl.ds(j * 16, 16)] = zvec

        @pl.loop(0, RPS // ZR)
        def _(t):
            pltpu.sync_copy(zero_v, acc_sh.at[pl.ds(s * RPS + t * ZR, ZR)])

        plsc.subcore_barrier()

        for j in range(3):
            pltpu.async_copy(er_hbm.at[wid, j], idx.at[j], isem[j])
        pltpu.make_async_copy(er_hbm.at[wid, 0], idx.at[0], isem[0]).wait()
        pltpu.async_copy(g_hbm.at[idx.at[0, 0]], rows.at[0], gsem[0])

        @pl.loop(0, CH, step=4)
        def _(i0):
            for u in range(4):
                i = i0 + u
                b = u % 2
                nb = (u + 1) % 2
                pltpu.make_async_copy(
                    g_hbm.at[idx.at[u, 0]], rows.at[b], gsem[b]
                ).wait()

                @pl.when(i >= 1)
                def _():
                    pltpu.make_async_copy(
                        rows.at[nb], acc_sh.at[idx.at[(u + 3) % 4, 1]],
                        ssem[nb],
                    ).wait()

                @pl.when(i + 1 < CH)
                def _():
                    pltpu.make_async_copy(
                        er_hbm.at[wid, i + 1], idx.at[(u + 1) % 4],
                        isem[(u + 1) % 4],
                    ).wait()
                    pltpu.async_copy(
                        g_hbm.at[idx.at[(u + 1) % 4, 0]], rows.at[nb],
                        gsem[nb],
                    )

                pltpu.async_copy(
                    rows.at[b], acc_sh.at[idx.at[u, 1]], ssem[b], add=True
                )

                @pl.when(i + 3 < CH)
                def _():
                    pltpu.async_copy(
                        er_hbm.at[wid, i + 3], idx.at[(u + 3) % 4],
                        isem[(u + 3) % 4],
                    )

        pltpu.make_async_copy(
            rows.at[(CH - 1) % 2], acc_sh.at[idx.at[(CH - 1) % 4, 1]],
            ssem[(CH - 1) % 2],
        ).wait()

        plsc.subcore_barrier()
        pltpu.sync_copy(
            acc_sh.at[pl.ds(s * RPS, RPS)], out_hbm.at[c, pl.ds(s * RPS, RPS)]
        )

    return k(g, er)


def _dinv_col(degp_a, degp_b):
    return jax.lax.rsqrt(degp_a + degp_b + 1.0)[:, 0:1]


def _mm(x, W):
    def body(x_ref, w_ref, o_ref):
        o_ref[...] = jnp.dot(x_ref[...], w_ref[...], preferred_element_type=_f32)

    return pl.pallas_call(
        body, out_shape=jax.ShapeDtypeStruct((N, D), _f32)
    )(x, W)


def _scale(h, degp):
    def body(h_ref, degp_ref, o_ref):
        dinv = _dinv_col(degp_ref[0, :N], degp_ref[1, :N])
        o_ref[...] = h_ref[...] * dinv

    return pl.pallas_call(
        body, out_shape=jax.ShapeDtypeStruct((N, D), _f32)
    )(h, degp)


def _mid(p, g1, degp, b1, W2):
    def body(p_ref, g1_ref, degp_ref, b_ref, w_ref, o_ref):
        dinv = _dinv_col(degp_ref[0, :N], degp_ref[1, :N])
        x2 = jnp.maximum(
            dinv * (p_ref[0, :N] + p_ref[1, :N] + g1_ref[...]) + b_ref[...], 0.0
        )
        o_ref[...] = jnp.dot(x2, w_ref[...], preferred_element_type=_f32) * dinv

    return pl.pallas_call(
        body, out_shape=jax.ShapeDtypeStruct((N, D), _f32)
    )(p, g1, degp, b1.reshape(1, D), W2)


def _final(p, g2, degp, b2):
    def body(p_ref, g2_ref, degp_ref, b_ref, o_ref):
        dinv = _dinv_col(degp_ref[0, :N], degp_ref[1, :N])
        o_ref[...] = jnp.maximum(
            dinv * (p_ref[0, :N] + p_ref[1, :N] + g2_ref[...]) + b_ref[...], 0.0
        )

    return pl.pallas_call(
        body, out_shape=jax.ShapeDtypeStruct((N, D), _f32)
    )(p, g2, degp, b2.reshape(1, D))


def kernel(x, edge_index, W1, b1, W2, b2):
    er = _edges_interleaved(edge_index[0], edge_index[1])

    degp = _deg_partials(er)
    h1 = _mm(x, W1)
    g1 = _scale(h1, degp)
    p1 = _msg_partials(g1, er)
    g2 = _mid(p1, g1, degp, b1, W2)
    p2 = _msg_partials(g2, er)
    out = _final(p2, g2, degp, b2)
    return out


# speedup vs baseline: 27.7482x; 1.1161x over previous
import functools

import jax
import jax.numpy as jnp
from jax import lax
from jax.experimental import pallas as pl
from jax.experimental.pallas import tpu as pltpu
from jax.experimental.pallas import tpu_sc as plsc

N = 10000
NP = 10112
E = 320000
D = 128

NC = 2
NS = 16
NW = NC * NS
K = 64
CH = 160
EPAD = NW * CH * K
FB = 8
RPS = NP // NS
ZR = 8

_f32 = jnp.float32


def _sc_mesh():
    return plsc.VectorSubcoreMesh(core_axis_name="c", subcore_axis_name="s")


def _edges_interleaved(src, dst):
    pad = EPAD - E
    ar = jnp.arange(pad, dtype=jnp.int32)
    src_f = jnp.concatenate([src, (ar * 131) % N])
    dst_f = jnp.concatenate([dst, N + (ar % (NP - N))])
    return jnp.stack(
        [src_f.reshape(NW, CH, K), dst_f.reshape(NW, CH, K)], axis=2
    )


def _deg_partials(er):
    @functools.partial(
        pl.kernel,
        out_type=jax.ShapeDtypeStruct((NC, NP, D), _f32),
        mesh=_sc_mesh(),
        scratch_types=[
            pltpu.VMEM((CH, 2, K), jnp.int32),
            pltpu.VMEM((K, D), _f32),
            pltpu.VMEM((ZR, D), _f32),
            pltpu.VMEM_SHARED((NP, D), _f32),
            pltpu.SemaphoreType.DMA,
        ],
    )
    def k(er_hbm, out_hbm, idx_all, ones_v, zero_v, acc_sh, sem):
        c = lax.axis_index("c")
        s = lax.axis_index("s")
        wid = c * NS + s
        pltpu.sync_copy(er_hbm.at[wid], idx_all)
        zvec = jnp.zeros((16,), _f32)
        ovec = jnp.full((16,), 1.0, _f32)

        @pl.loop(0, ZR)
        def _(r):
            @pl.loop(0, D // 16)
            def _(j):
                zero_v[r, pl.ds(j * 16, 16)] = zvec

        @pl.loop(0, K)
        def _(r):
            @pl.loop(0, D // 16)
            def _(j):
                ones_v[r, pl.ds(j * 16, 16)] = ovec

        @pl.loop(0, RPS // ZR)
        def _(t):
            pltpu.sync_copy(zero_v, acc_sh.at[pl.ds(s * RPS + t * ZR, ZR)])

        plsc.subcore_barrier()

        @pl.loop(0, CH, step=FB)
        def _(i0):
            for b in range(FB):
                pltpu.async_copy(
                    ones_v, acc_sh.at[idx_all.at[i0 + b, 1]], sem, add=True
                )
            for b in range(FB):
                pltpu.make_async_copy(
                    ones_v, acc_sh.at[idx_all.at[i0 + b, 1]], sem
                ).wait()

        plsc.subcore_barrier()
        pltpu.sync_copy(
            acc_sh.at[pl.ds(s * RPS, RPS)], out_hbm.at[c, pl.ds(s * RPS, RPS)]
        )

    return k(er)


def _msg_partials(g, er):
    @functools.partial(
        pl.kernel,
        out_type=jax.ShapeDtypeStruct((NC, NP, D), _f32),
        mesh=_sc_mesh(),
        scratch_types=[
            pltpu.VMEM((8, 2, K), jnp.int32),
            pltpu.VMEM((4, K, D), _f32),
            pltpu.VMEM((ZR, D), _f32),
            pltpu.VMEM_SHARED((NP, D), _f32),
        ] + [pltpu.SemaphoreType.DMA] * 16,
    )
    def k(g_hbm, er_hbm, out_hbm, idx, rows, zero_v, acc_sh, *sems):
        gsem = sems[:4]
        ssem = sems[4:8]
        isem = sems[8:16]
        c = lax.axis_index("c")
        s = lax.axis_index("s")
        wid = c * NS + s
        zvec = jnp.zeros((16,), _f32)

        @pl.loop(0, ZR)
        def _(r):
            @pl.loop(0, D // 16)
            def _(j):
                zero_v[r, pl.ds(j * 16, 16)] = zvec

        @pl.loop(0, RPS // ZR)
        def _(t):
            pltpu.sync_copy(zero_v, acc_sh.at[pl.ds(s * RPS + t * ZR, ZR)])

        plsc.subcore_barrier()

        for j in range(6):
            pltpu.async_copy(er_hbm.at[wid, j], idx.at[j], isem[j])
        for j in range(2):
            pltpu.make_async_copy(er_hbm.at[wid, j], idx.at[j], isem[j]).wait()
            pltpu.async_copy(g_hbm.at[idx.at[j, 0]], rows.at[j], gsem[j])

        @pl.loop(0, CH, step=8)
        def _(i0):
            for u in range(8):
                i = i0 + u
                r = u % 4
                nr = (u + 2) % 4
                @pl.when(i >= 2)
                def _():
                    pltpu.make_async_copy(
                        rows.at[nr], acc_sh.at[idx.at[(u + 6) % 8, 1]],
                        ssem[nr],
                    ).wait()

                @pl.when(i + 6 < CH)
                def _():
                    pltpu.async_copy(
                        er_hbm.at[wid, i + 6], idx.at[(u + 6) % 8],
                        isem[(u + 6) % 8],
                    )

                @pl.when(i + 2 < CH)
                def _():
                    pltpu.make_async_copy(
                        er_hbm.at[wid, i + 2], idx.at[(u + 2) % 8],
                        isem[(u + 2) % 8],
                    ).wait()
                    pltpu.async_copy(
                        g_hbm.at[idx.at[(u + 2) % 8, 0]], rows.at[nr],
                        gsem[nr],
                    )

                pltpu.make_async_copy(
                    g_hbm.at[idx.at[u, 0]], rows.at[r], gsem[r]
                ).wait()

                pltpu.async_copy(
                    rows.at[r], acc_sh.at[idx.at[u, 1]], ssem[r], add=True
                )

        for i in (CH - 2, CH - 1):
            pltpu.make_async_copy(
                rows.at[i % 4], acc_sh.at[idx.at[i % 8, 1]], ssem[i % 4]
            ).wait()

        plsc.subcore_barrier()
        pltpu.sync_copy(
            acc_sh.at[pl.ds(s * RPS, RPS)], out_hbm.at[c, pl.ds(s * RPS, RPS)]
        )

    return k(g, er)


def _dinv_col(degp_a, degp_b):
    return jax.lax.rsqrt(degp_a + degp_b + 1.0)[:, 0:1]


def _mm(x, W):
    def body(x_ref, w_ref, o_ref):
        o_ref[...] = jnp.dot(x_ref[...], w_ref[...], preferred_element_type=_f32)

    return pl.pallas_call(
        body, out_shape=jax.ShapeDtypeStruct((N, D), _f32)
    )(x, W)


def _scale(h, degp):
    def body(h_ref, degp_ref, o_ref):
        dinv = _dinv_col(degp_ref[0, :N], degp_ref[1, :N])
        o_ref[...] = h_ref[...] * dinv

    return pl.pallas_call(
        body, out_shape=jax.ShapeDtypeStruct((N, D), _f32)
    )(h, degp)


def _mid(p, g1, degp, b1, W2):
    def body(p_ref, g1_ref, degp_ref, b_ref, w_ref, o_ref):
        dinv = _dinv_col(degp_ref[0, :N], degp_ref[1, :N])
        x2 = jnp.maximum(
            dinv * (p_ref[0, :N] + p_ref[1, :N] + g1_ref[...]) + b_ref[...], 0.0
        )
        o_ref[...] = jnp.dot(x2, w_ref[...], preferred_element_type=_f32) * dinv

    return pl.pallas_call(
        body, out_shape=jax.ShapeDtypeStruct((N, D), _f32)
    )(p, g1, degp, b1.reshape(1, D), W2)


def _final(p, g2, degp, b2):
    def body(p_ref, g2_ref, degp_ref, b_ref, o_ref):
        dinv = _dinv_col(degp_ref[0, :N], degp_ref[1, :N])
        o_ref[...] = jnp.maximum(
            dinv * (p_ref[0, :N] + p_ref[1, :N] + g2_ref[...]) + b_ref[...], 0.0
        )

    return pl.pallas_call(
        body, out_shape=jax.ShapeDtypeStruct((N, D), _f32)
    )(p, g2, degp, b2.reshape(1, D))


def kernel(x, edge_index, W1, b1, W2, b2):
    er = _edges_interleaved(edge_index[0], edge_index[1])

    degp = _deg_partials(er)
    h1 = _mm(x, W1)
    g1 = _scale(h1, degp)
    p1 = _msg_partials(g1, er)
    g2 = _mid(p1, g1, degp, b1, W2)
    p2 = _msg_partials(g2, er)
    out = _final(p2, g2, degp, b2)
    return out


# speedup vs baseline: 29.6333x; 1.0679x over previous
import functools

import jax
import jax.numpy as jnp
from jax import lax
from jax.experimental import pallas as pl
from jax.experimental.pallas import tpu as pltpu
from jax.experimental.pallas import tpu_sc as plsc

N = 10000
NP = 10112
E = 320000
D = 128

NC = 2
NS = 16
NW = NC * NS
K = 64
CH = 160
EPAD = NW * CH * K
FB = 8
RPS = NP // NS
ZR = 8

_f32 = jnp.float32


def _sc_mesh():
    return plsc.VectorSubcoreMesh(core_axis_name="c", subcore_axis_name="s")


def _edges_interleaved(src, dst):
    pad = EPAD - E
    ar = jnp.arange(pad, dtype=jnp.int32)
    src_f = jnp.concatenate([src, (ar * 131) % N])
    dst_f = jnp.concatenate([dst, N + (ar % (NP - N))])
    return jnp.stack(
        [src_f.reshape(NW, CH, K), dst_f.reshape(NW, CH, K)], axis=2
    )


def _deg_partials(er):
    @functools.partial(
        pl.kernel,
        out_type=jax.ShapeDtypeStruct((NC, NP, D), _f32),
        mesh=_sc_mesh(),
        scratch_types=[
            pltpu.VMEM((CH, 2, K), jnp.int32),
            pltpu.VMEM((K, D), _f32),
            pltpu.VMEM((ZR, D), _f32),
            pltpu.VMEM_SHARED((NP, D), _f32),
            pltpu.SemaphoreType.DMA,
        ],
    )
    def k(er_hbm, out_hbm, idx_all, ones_v, zero_v, acc_sh, sem):
        c = lax.axis_index("c")
        s = lax.axis_index("s")
        wid = c * NS + s
        pltpu.sync_copy(er_hbm.at[wid], idx_all)
        zvec = jnp.zeros((16,), _f32)
        ovec = jnp.full((16,), 1.0, _f32)

        @pl.loop(0, ZR)
        def _(r):
            @pl.loop(0, D // 16)
            def _(j):
                zero_v[r, pl.ds(j * 16, 16)] = zvec

        @pl.loop(0, K)
        def _(r):
            @pl.loop(0, D // 16)
            def _(j):
                ones_v[r, pl.ds(j * 16, 16)] = ovec

        @pl.loop(0, RPS // ZR)
        def _(t):
            pltpu.sync_copy(zero_v, acc_sh.at[pl.ds(s * RPS + t * ZR, ZR)])

        plsc.subcore_barrier()

        @pl.loop(0, CH, step=FB)
        def _(i0):
            for b in range(FB):
                pltpu.async_copy(
                    ones_v, acc_sh.at[idx_all.at[i0 + b, 1]], sem, add=True
                )
            for b in range(FB):
                pltpu.make_async_copy(
                    ones_v, acc_sh.at[idx_all.at[i0 + b, 1]], sem
                ).wait()

        plsc.subcore_barrier()
        pltpu.sync_copy(
            acc_sh.at[pl.ds(s * RPS, RPS)], out_hbm.at[c, pl.ds(s * RPS, RPS)]
        )

    return k(er)


def _msg_partials(g, er):
    @functools.partial(
        pl.kernel,
        out_type=jax.ShapeDtypeStruct((NC, NP, D), _f32),
        mesh=_sc_mesh(),
        scratch_types=[
            pltpu.VMEM((8, 2, K), jnp.int32),
            pltpu.VMEM((4, K, D), _f32),
            pltpu.VMEM((ZR, D), _f32),
            pltpu.VMEM_SHARED((NP, D), _f32),
        ] + [pltpu.SemaphoreType.DMA] * 16,
    )
    def k(g_hbm, er_hbm, out_hbm, idx, rows, zero_v, acc_sh, *sems):
        gsem = sems[:4]
        ssem = sems[4:8]
        isem = sems[8:16]
        c = lax.axis_index("c")
        s = lax.axis_index("s")
        wid = c * NS + s
        zvec = jnp.zeros((16,), _f32)

        @pl.loop(0, ZR)
        def _(r):
            @pl.loop(0, D // 16)
            def _(j):
                zero_v[r, pl.ds(j * 16, 16)] = zvec

        @pl.loop(0, RPS // ZR)
        def _(t):
            pltpu.sync_copy(zero_v, acc_sh.at[pl.ds(s * RPS + t * ZR, ZR)])

        plsc.subcore_barrier()

        for j in range(7):
            pltpu.async_copy(er_hbm.at[wid, j], idx.at[j], isem[j])
        for j in range(3):
            pltpu.make_async_copy(er_hbm.at[wid, j], idx.at[j], isem[j]).wait()
            pltpu.async_copy(g_hbm.at[idx.at[j, 0]], rows.at[j], gsem[j])

        @pl.loop(0, CH, step=8)
        def _(i0):
            for u in range(8):
                i = i0 + u
                r = u % 4
                nr = (u + 3) % 4
                @pl.when(i >= 1)
                def _():
                    pltpu.make_async_copy(
                        rows.at[nr], acc_sh.at[idx.at[(u + 7) % 8, 1]],
                        ssem[nr],
                    ).wait()

                @pl.when(i + 7 < CH)
                def _():
                    pltpu.async_copy(
                        er_hbm.at[wid, i + 7], idx.at[(u + 7) % 8],
                        isem[(u + 7) % 8],
                    )

                @pl.when(i + 3 < CH)
                def _():
                    pltpu.make_async_copy(
                        er_hbm.at[wid, i + 3], idx.at[(u + 3) % 8],
                        isem[(u + 3) % 8],
                    ).wait()
                    pltpu.async_copy(
                        g_hbm.at[idx.at[(u + 3) % 8, 0]], rows.at[nr],
                        gsem[nr],
                    )

                pltpu.make_async_copy(
                    g_hbm.at[idx.at[u, 0]], rows.at[r], gsem[r]
                ).wait()

                pltpu.async_copy(
                    rows.at[r], acc_sh.at[idx.at[u, 1]], ssem[r], add=True
                )

        pltpu.make_async_copy(
            rows.at[(CH - 1) % 4], acc_sh.at[idx.at[(CH - 1) % 8, 1]],
            ssem[(CH - 1) % 4],
        ).wait()

        plsc.subcore_barrier()
        pltpu.sync_copy(
            acc_sh.at[pl.ds(s * RPS, RPS)], out_hbm.at[c, pl.ds(s * RPS, RPS)]
        )

    return k(g, er)


def _dinv_col(degp_a, degp_b):
    return jax.lax.rsqrt(degp_a + degp_b + 1.0)[:, 0:1]


def _mm(x, W):
    def body(x_ref, w_ref, o_ref):
        o_ref[...] = jnp.dot(x_ref[...], w_ref[...], preferred_element_type=_f32)

    return pl.pallas_call(
        body, out_shape=jax.ShapeDtypeStruct((N, D), _f32)
    )(x, W)


def _scale(h, degp):
    def body(h_ref, degp_ref, o_ref):
        dinv = _dinv_col(degp_ref[0, :N], degp_ref[1, :N])
        o_ref[...] = h_ref[...] * dinv

    return pl.pallas_call(
        body, out_shape=jax.ShapeDtypeStruct((N, D), _f32)
    )(h, degp)


def _mid(p, g1, degp, b1, W2):
    def body(p_ref, g1_ref, degp_ref, b_ref, w_ref, o_ref):
        dinv = _dinv_col(degp_ref[0, :N], degp_ref[1, :N])
        x2 = jnp.maximum(
            dinv * (p_ref[0, :N] + p_ref[1, :N] + g1_ref[...]) + b_ref[...], 0.0
        )
        o_ref[...] = jnp.dot(x2, w_ref[...], preferred_element_type=_f32) * dinv

    return pl.pallas_call(
        body, out_shape=jax.ShapeDtypeStruct((N, D), _f32)
    )(p, g1, degp, b1.reshape(1, D), W2)


def _final(p, g2, degp, b2):
    def body(p_ref, g2_ref, degp_ref, b_ref, o_ref):
        dinv = _dinv_col(degp_ref[0, :N], degp_ref[1, :N])
        o_ref[...] = jnp.maximum(
            dinv * (p_ref[0, :N] + p_ref[1, :N] + g2_ref[...]) + b_ref[...], 0.0
        )

    return pl.pallas_call(
        body, out_shape=jax.ShapeDtypeStruct((N, D), _f32)
    )(p, g2, degp, b2.reshape(1, D))


def kernel(x, edge_index, W1, b1, W2, b2):
    er = _edges_interleaved(edge_index[0], edge_index[1])

    degp = _deg_partials(er)
    h1 = _mm(x, W1)
    g1 = _scale(h1, degp)
    p1 = _msg_partials(g1, er)
    g2 = _mid(p1, g1, degp, b1, W2)
    p2 = _msg_partials(g2, er)
    out = _final(p2, g2, degp, b2)
    return out


# speedup vs baseline: 31.2743x; 1.0554x over previous
import functools

import jax
import jax.numpy as jnp
from jax import lax
from jax.experimental import pallas as pl
from jax.experimental.pallas import tpu as pltpu
from jax.experimental.pallas import tpu_sc as plsc

N = 10000
NP = 10112
E = 320000
D = 128

NC = 2
NS = 16
NW = NC * NS
K = 64
CH = 160
EPAD = NW * CH * K
FB = 8
RPS = NP // NS
ZR = 8

_f32 = jnp.float32


def _sc_mesh():
    return plsc.VectorSubcoreMesh(core_axis_name="c", subcore_axis_name="s")


def _edges_interleaved(src, dst):
    pad = EPAD - E
    ar = jnp.arange(pad, dtype=jnp.int32)
    src_f = jnp.concatenate([src, (ar * 131) % N])
    dst_f = jnp.concatenate([dst, N + (ar % (NP - N))])
    return jnp.stack(
        [src_f.reshape(NW, CH, K), dst_f.reshape(NW, CH, K)], axis=2
    )


def _deg_partials(er):
    @functools.partial(
        pl.kernel,
        out_type=jax.ShapeDtypeStruct((NC, NP, D), _f32),
        mesh=_sc_mesh(),
        scratch_types=[
            pltpu.VMEM((CH, 2, K), jnp.int32),
            pltpu.VMEM((K, D), _f32),
            pltpu.VMEM((ZR, D), _f32),
            pltpu.VMEM_SHARED((NP, D), _f32),
            pltpu.SemaphoreType.DMA,
            pltpu.SemaphoreType.DMA,
            pltpu.SemaphoreType.DMA,
        ],
    )
    def k(er_hbm, out_hbm, idx_all, ones_v, zero_v, acc_sh, sem, zsem, lsem):
        c = lax.axis_index("c")
        s = lax.axis_index("s")
        wid = c * NS + s
        pltpu.async_copy(er_hbm.at[wid], idx_all, lsem)
        zvec = jnp.zeros((16,), _f32)
        ovec = jnp.full((16,), 1.0, _f32)

        @pl.loop(0, ZR)
        def _(r):
            @pl.loop(0, D // 16)
            def _(j):
                zero_v[r, pl.ds(j * 16, 16)] = zvec

        @pl.loop(0, RPS // ZR)
        def _(t):
            pltpu.async_copy(
                zero_v, acc_sh.at[pl.ds(s * RPS + t * ZR, ZR)], zsem
            )

        @pl.loop(0, K)
        def _(r):
            @pl.loop(0, D // 16)
            def _(j):
                ones_v[r, pl.ds(j * 16, 16)] = ovec

        @pl.loop(0, RPS // ZR)
        def _(t):
            pltpu.make_async_copy(
                zero_v, acc_sh.at[pl.ds(s * RPS + t * ZR, ZR)], zsem
            ).wait()

        pltpu.make_async_copy(er_hbm.at[wid], idx_all, lsem).wait()
        plsc.subcore_barrier()

        @pl.loop(0, CH, step=FB)
        def _(i0):
            for b in range(FB):
                pltpu.async_copy(
                    ones_v, acc_sh.at[idx_all.at[i0 + b, 1]], sem, add=True
                )
            for b in range(FB):
                pltpu.make_async_copy(
                    ones_v, acc_sh.at[idx_all.at[i0 + b, 1]], sem
                ).wait()

        plsc.subcore_barrier()
        pltpu.sync_copy(
            acc_sh.at[pl.ds(s * RPS, RPS)], out_hbm.at[c, pl.ds(s * RPS, RPS)]
        )

    return k(er)


def _msg_partials(g, er):
    @functools.partial(
        pl.kernel,
        out_type=jax.ShapeDtypeStruct((NC, NP, D), _f32),
        mesh=_sc_mesh(),
        scratch_types=[
            pltpu.VMEM((8, 2, K), jnp.int32),
            pltpu.VMEM((4, K, D), _f32),
            pltpu.VMEM((ZR, D), _f32),
            pltpu.VMEM_SHARED((NP, D), _f32),
        ] + [pltpu.SemaphoreType.DMA] * 17,
    )
    def k(g_hbm, er_hbm, out_hbm, idx, rows, zero_v, acc_sh, *sems):
        gsem = sems[:4]
        ssem = sems[4:8]
        isem = sems[8:16]
        zsem = sems[16]
        c = lax.axis_index("c")
        s = lax.axis_index("s")
        wid = c * NS + s
        zvec = jnp.zeros((16,), _f32)

        @pl.loop(0, ZR)
        def _(r):
            @pl.loop(0, D // 16)
            def _(j):
                zero_v[r, pl.ds(j * 16, 16)] = zvec

        @pl.loop(0, RPS // ZR)
        def _(t):
            pltpu.async_copy(
                zero_v, acc_sh.at[pl.ds(s * RPS + t * ZR, ZR)], zsem
            )

        for j in range(7):
            pltpu.async_copy(er_hbm.at[wid, j], idx.at[j], isem[j])
        for j in range(3):
            pltpu.make_async_copy(er_hbm.at[wid, j], idx.at[j], isem[j]).wait()
            pltpu.async_copy(g_hbm.at[idx.at[j, 0]], rows.at[j], gsem[j])

        @pl.loop(0, RPS // ZR)
        def _(t):
            pltpu.make_async_copy(
                zero_v, acc_sh.at[pl.ds(s * RPS + t * ZR, ZR)], zsem
            ).wait()

        plsc.subcore_barrier()

        @pl.loop(0, CH, step=8)
        def _(i0):
            for u in range(8):
                i = i0 + u
                r = u % 4
                nr = (u + 3) % 4
                @pl.when(i >= 1)
                def _():
                    pltpu.make_async_copy(
                        rows.at[nr], acc_sh.at[idx.at[(u + 7) % 8, 1]],
                        ssem[nr],
                    ).wait()

                @pl.when(i + 7 < CH)
                def _():
                    pltpu.async_copy(
                        er_hbm.at[wid, i + 7], idx.at[(u + 7) % 8],
                        isem[(u + 7) % 8],
                    )

                @pl.when(i + 3 < CH)
                def _():
                    pltpu.make_async_copy(
                        er_hbm.at[wid, i + 3], idx.at[(u + 3) % 8],
                        isem[(u + 3) % 8],
                    ).wait()
                    pltpu.async_copy(
                        g_hbm.at[idx.at[(u + 3) % 8, 0]], rows.at[nr],
                        gsem[nr],
                    )

                pltpu.make_async_copy(
                    g_hbm.at[idx.at[u, 0]], rows.at[r], gsem[r]
                ).wait()

                pltpu.async_copy(
                    rows.at[r], acc_sh.at[idx.at[u, 1]], ssem[r], add=True
                )

        pltpu.make_async_copy(
            rows.at[(CH - 1) % 4], acc_sh.at[idx.at[(CH - 1) % 8, 1]],
            ssem[(CH - 1) % 4],
        ).wait()

        plsc.subcore_barrier()
        pltpu.sync_copy(
            acc_sh.at[pl.ds(s * RPS, RPS)], out_hbm.at[c, pl.ds(s * RPS, RPS)]
        )

    return k(g, er)


def _dinv_col(degp_a, degp_b):
    return jax.lax.rsqrt(degp_a + degp_b + 1.0)[:, 0:1]


def _mm(x, W):
    def body(x_ref, w_ref, o_ref):
        o_ref[...] = jnp.dot(x_ref[...], w_ref[...], preferred_element_type=_f32)

    return pl.pallas_call(
        body, out_shape=jax.ShapeDtypeStruct((N, D), _f32)
    )(x, W)


def _scale(h, degp):
    def body(h_ref, degp_ref, o_ref):
        dinv = _dinv_col(degp_ref[0, :N], degp_ref[1, :N])
        o_ref[...] = h_ref[...] * dinv

    return pl.pallas_call(
        body, out_shape=jax.ShapeDtypeStruct((N, D), _f32)
    )(h, degp)


def _mid(p, g1, degp, b1, W2):
    def body(p_ref, g1_ref, degp_ref, b_ref, w_ref, o_ref):
        dinv = _dinv_col(degp_ref[0, :N], degp_ref[1, :N])
        x2 = jnp.maximum(
            dinv * (p_ref[0, :N] + p_ref[1, :N] + g1_ref[...]) + b_ref[...], 0.0
        )
        o_ref[...] = jnp.dot(x2, w_ref[...], preferred_element_type=_f32) * dinv

    return pl.pallas_call(
        body, out_shape=jax.ShapeDtypeStruct((N, D), _f32)
    )(p, g1, degp, b1.reshape(1, D), W2)


def _final(p, g2, degp, b2):
    def body(p_ref, g2_ref, degp_ref, b_ref, o_ref):
        dinv = _dinv_col(degp_ref[0, :N], degp_ref[1, :N])
        o_ref[...] = jnp.maximum(
            dinv * (p_ref[0, :N] + p_ref[1, :N] + g2_ref[...]) + b_ref[...], 0.0
        )

    return pl.pallas_call(
        body, out_shape=jax.ShapeDtypeStruct((N, D), _f32)
    )(p, g2, degp, b2.reshape(1, D))


def kernel(x, edge_index, W1, b1, W2, b2):
    er = _edges_interleaved(edge_index[0], edge_index[1])

    degp = _deg_partials(er)
    h1 = _mm(x, W1)
    g1 = _scale(h1, degp)
    p1 = _msg_partials(g1, er)
    g2 = _mid(p1, g1, degp, b1, W2)
    p2 = _msg_partials(g2, er)
    out = _final(p2, g2, degp, b2)
    return out


# speedup vs baseline: 31.2991x; 1.0008x over previous
import functools

import jax
import jax.numpy as jnp
from jax import lax
from jax.experimental import pallas as pl
from jax.experimental.pallas import tpu as pltpu
from jax.experimental.pallas import tpu_sc as plsc

N = 10000
NP = 10112
E = 320000
D = 128

NC = 2
NS = 16
NW = NC * NS
K = 64
CH = 160
EPAD = NW * CH * K
FB = 8
RPS = NP // NS
ZR = 8

_f32 = jnp.float32


def _sc_mesh():
    return plsc.VectorSubcoreMesh(core_axis_name="c", subcore_axis_name="s")


def _edges_interleaved(src, dst):
    pad = EPAD - E
    ar = jnp.arange(pad, dtype=jnp.int32)
    src_f = jnp.concatenate([src, (ar * 131) % N])
    dst_f = jnp.concatenate([dst, N + (ar % (NP - N))])
    return jnp.stack(
        [src_f.reshape(NW, CH, K), dst_f.reshape(NW, CH, K)], axis=2
    )


def _deg_partials(er):
    @functools.partial(
        pl.kernel,
        out_type=jax.ShapeDtypeStruct((NC, NP, D), _f32),
        mesh=_sc_mesh(),
        scratch_types=[
            pltpu.VMEM((CH, 2, K), jnp.int32),
            pltpu.VMEM((K, D), _f32),
            pltpu.VMEM((ZR, D), _f32),
            pltpu.VMEM_SHARED((NP, D), _f32),
            pltpu.SemaphoreType.DMA,
            pltpu.SemaphoreType.DMA,
            pltpu.SemaphoreType.DMA,
        ],
    )
    def k(er_hbm, out_hbm, idx_all, ones_v, zero_v, acc_sh, sem, zsem, lsem):
        c = lax.axis_index("c")
        s = lax.axis_index("s")
        wid = c * NS + s
        pltpu.async_copy(er_hbm.at[wid], idx_all, lsem)
        zvec = jnp.zeros((16,), _f32)
        ovec = jnp.full((16,), 1.0, _f32)

        @pl.loop(0, ZR)
        def _(r):
            @pl.loop(0, D // 16)
            def _(j):
                zero_v[r, pl.ds(j * 16, 16)] = zvec

        @pl.loop(0, RPS // ZR)
        def _(t):
            pltpu.async_copy(
                zero_v, acc_sh.at[pl.ds(s * RPS + t * ZR, ZR)], zsem
            )

        @pl.loop(0, K)
        def _(r):
            @pl.loop(0, D // 16)
            def _(j):
                ones_v[r, pl.ds(j * 16, 16)] = ovec

        @pl.loop(0, RPS // ZR)
        def _(t):
            pltpu.make_async_copy(
                zero_v, acc_sh.at[pl.ds(s * RPS + t * ZR, ZR)], zsem
            ).wait()

        pltpu.make_async_copy(er_hbm.at[wid], idx_all, lsem).wait()
        plsc.subcore_barrier()

        @pl.loop(0, CH)
        def _(i):
            pltpu.async_copy(
                ones_v, acc_sh.at[idx_all.at[i, 1]], sem, add=True
            )

            @pl.when(i >= FB)
            def _():
                pltpu.make_async_copy(
                    ones_v, acc_sh.at[idx_all.at[i - FB, 1]], sem
                ).wait()

        @pl.loop(CH - FB, CH)
        def _(i):
            pltpu.make_async_copy(
                ones_v, acc_sh.at[idx_all.at[i, 1]], sem
            ).wait()

        plsc.subcore_barrier()
        pltpu.sync_copy(
            acc_sh.at[pl.ds(s * RPS, RPS)], out_hbm.at[c, pl.ds(s * RPS, RPS)]
        )

    return k(er)


def _msg_partials(g, er):
    @functools.partial(
        pl.kernel,
        out_type=jax.ShapeDtypeStruct((NC, NP, D), _f32),
        mesh=_sc_mesh(),
        scratch_types=[
            pltpu.VMEM((8, 2, K), jnp.int32),
            pltpu.VMEM((4, K, D), _f32),
            pltpu.VMEM((ZR, D), _f32),
            pltpu.VMEM_SHARED((NP, D), _f32),
        ] + [pltpu.SemaphoreType.DMA] * 17,
    )
    def k(g_hbm, er_hbm, out_hbm, idx, rows, zero_v, acc_sh, *sems):
        gsem = sems[:4]
        ssem = sems[4:8]
        isem = sems[8:16]
        zsem = sems[16]
        c = lax.axis_index("c")
        s = lax.axis_index("s")
        wid = c * NS + s
        zvec = jnp.zeros((16,), _f32)

        @pl.loop(0, ZR)
        def _(r):
            @pl.loop(0, D // 16)
            def _(j):
                zero_v[r, pl.ds(j * 16, 16)] = zvec

        @pl.loop(0, RPS // ZR)
        def _(t):
            pltpu.async_copy(
                zero_v, acc_sh.at[pl.ds(s * RPS + t * ZR, ZR)], zsem
            )

        for j in range(7):
            pltpu.async_copy(er_hbm.at[wid, j], idx.at[j], isem[j])
        for j in range(3):
            pltpu.make_async_copy(er_hbm.at[wid, j], idx.at[j], isem[j]).wait()
            pltpu.async_copy(g_hbm.at[idx.at[j, 0]], rows.at[j], gsem[j])

        @pl.loop(0, RPS // ZR)
        def _(t):
            pltpu.make_async_copy(
                zero_v, acc_sh.at[pl.ds(s * RPS + t * ZR, ZR)], zsem
            ).wait()

        plsc.subcore_barrier()

        @pl.loop(0, CH, step=8)
        def _(i0):
            for u in range(8):
                i = i0 + u
                r = u % 4
                nr = (u + 3) % 4
                @pl.when(i >= 1)
                def _():
                    pltpu.make_async_copy(
                        rows.at[nr], acc_sh.at[idx.at[(u + 7) % 8, 1]],
                        ssem[nr],
                    ).wait()

                @pl.when(i + 7 < CH)
                def _():
                    pltpu.async_copy(
                        er_hbm.at[wid, i + 7], idx.at[(u + 7) % 8],
                        isem[(u + 7) % 8],
                    )

                @pl.when(i + 3 < CH)
                def _():
                    pltpu.make_async_copy(
                        er_hbm.at[wid, i + 3], idx.at[(u + 3) % 8],
                        isem[(u + 3) % 8],
                    ).wait()
                    pltpu.async_copy(
                        g_hbm.at[idx.at[(u + 3) % 8, 0]], rows.at[nr],
                        gsem[nr],
                    )

                pltpu.make_async_copy(
                    g_hbm.at[idx.at[u, 0]], rows.at[r], gsem[r]
                ).wait()

                pltpu.async_copy(
                    rows.at[r], acc_sh.at[idx.at[u, 1]], ssem[r], add=True
                )

        pltpu.make_async_copy(
            rows.at[(CH - 1) % 4], acc_sh.at[idx.at[(CH - 1) % 8, 1]],
            ssem[(CH - 1) % 4],
        ).wait()

        plsc.subcore_barrier()
        pltpu.sync_copy(
            acc_sh.at[pl.ds(s * RPS, RPS)], out_hbm.at[c, pl.ds(s * RPS, RPS)]
        )

    return k(g, er)


def _dinv_col(degp_a, degp_b):
    return jax.lax.rsqrt(degp_a + degp_b + 1.0)[:, 0:1]


def _mm(x, W):
    def body(x_ref, w_ref, o_ref):
        o_ref[...] = jnp.dot(x_ref[...], w_ref[...], preferred_element_type=_f32)

    return pl.pallas_call(
        body, out_shape=jax.ShapeDtypeStruct((N, D), _f32)
    )(x, W)


def _scale(h, degp):
    def body(h_ref, degp_ref, o_ref):
        dinv = _dinv_col(degp_ref[0, :N], degp_ref[1, :N])
        o_ref[...] = h_ref[...] * dinv

    return pl.pallas_call(
        body, out_shape=jax.ShapeDtypeStruct((N, D), _f32)
    )(h, degp)


def _mid(p, g1, degp, b1, W2):
    def body(p_ref, g1_ref, degp_ref, b_ref, w_ref, o_ref):
        dinv = _dinv_col(degp_ref[0, :N], degp_ref[1, :N])
        x2 = jnp.maximum(
            dinv * (p_ref[0, :N] + p_ref[1, :N] + g1_ref[...]) + b_ref[...], 0.0
        )
        o_ref[...] = jnp.dot(x2, w_ref[...], preferred_element_type=_f32) * dinv

    return pl.pallas_call(
        body, out_shape=jax.ShapeDtypeStruct((N, D), _f32)
    )(p, g1, degp, b1.reshape(1, D), W2)


def _final(p, g2, degp, b2):
    def body(p_ref, g2_ref, degp_ref, b_ref, o_ref):
        dinv = _dinv_col(degp_ref[0, :N], degp_ref[1, :N])
        o_ref[...] = jnp.maximum(
            dinv * (p_ref[0, :N] + p_ref[1, :N] + g2_ref[...]) + b_ref[...], 0.0
        )

    return pl.pallas_call(
        body, out_shape=jax.ShapeDtypeStruct((N, D), _f32)
    )(p, g2, degp, b2.reshape(1, D))


def kernel(x, edge_index, W1, b1, W2, b2):
    er = _edges_interleaved(edge_index[0], edge_index[1])

    degp = _deg_partials(er)
    h1 = _mm(x, W1)
    g1 = _scale(h1, degp)
    p1 = _msg_partials(g1, er)
    g2 = _mid(p1, g1, degp, b1, W2)
    p2 = _msg_partials(g2, er)
    out = _final(p2, g2, degp, b2)
    return out
